# baseline (speedup 1.0000x reference)
import functools
import math
import jax
import jax.numpy as jnp
from jax.experimental import pallas as pl

_UB = 8


def _main(enc_ref, dec_ref, idxp_ref, idxc_ref, embs_ref,
          wqa_ref, bqa_ref, wqs_ref, bqs_ref, wk_ref, bk_ref,
          wd_ref, bd_ref, ptr_ref, h_ref, db_ref, *, C):
    T = enc_ref.shape[1]
    cp = idxp_ref.shape[2]
    V1, Dh = embs_ref.shape
    V = V1 - 1
    A = wk_ref.shape[1]
    R = _UB * cp
    bf16 = jnp.bfloat16
    f32 = jnp.float32
    inv_sqrt_a = 1.0 / math.sqrt(A)

    embs = embs_ref[...]
    embs_bf = embs.astype(bf16)

    qac = jnp.dot(enc_ref[0].astype(bf16), wqa_ref[...].astype(bf16),
                  preferred_element_type=f32) + bqa_ref[...]

    dec8 = dec_ref[0]
    onehot_dec = (dec8 == jax.lax.broadcasted_iota(jnp.int32, (_UB, V), 1)
                  ).astype(f32).astype(bf16)
    semantic = jnp.dot(onehot_dec, embs_bf[:V, :],
                       preferred_element_type=f32)
    qse8 = jnp.dot(semantic.astype(bf16), wqs_ref[...].astype(bf16),
                   preferred_element_type=f32) + bqs_ref[...]

    idx_col = idxc_ref[0].reshape(R, 1)
    idx_pad = idxp_ref[0]
    c_col = jax.lax.broadcasted_iota(jnp.int32, (_UB, cp, 1),
                                     1).reshape(R, 1)
    valid = (c_col < C) & (idx_col >= 0)

    iota_v = jax.lax.broadcasted_iota(jnp.int32, (R, V1), 1)
    G_f = jnp.where(idx_col == iota_v, 1.0, 0.0)
    idx_row_exp = jnp.broadcast_to(idx_pad[:, None, :],
                                   (_UB, cp, cp)).reshape(R, cp)
    lane_c = jax.lax.broadcasted_iota(jnp.int32, (R, cp), 1)
    has_later = jnp.any((idx_col == idx_row_exp) & (lane_c > c_col),
                        axis=1, keepdims=True)
    G = G_f.astype(bf16)
    S = jnp.where(has_later | ~valid, 0.0, G_f).astype(bf16)

    values = jnp.dot(G, embs_bf, preferred_element_type=f32)
    values_bf = values.astype(bf16)
    keys = jnp.dot(values_bf, wk_ref[...].astype(bf16),
                   preferred_element_type=f32) + bk_ref[...]

    qse_exp = jnp.broadcast_to(qse8[:, None, :],
                               (_UB, cp, A)).reshape(R, A)
    rowdot = jnp.sum(qse_exp * keys, axis=1, keepdims=True)
    pen = jnp.where(valid, 0.0, -1.0e9)
    k_aug = jnp.concatenate([keys, rowdot, pen],
                            axis=1).astype(bf16)
    ones_t = jnp.full((T, 2), 1.0, dtype=f32)
    q_aug = jnp.concatenate([qac, ones_t], axis=1).astype(bf16)
    qk = jax.lax.dot_general(q_aug, k_aug, (((1,), (1,)), ((), ())),
                             preferred_element_type=f32)
    logits = qk * inv_sqrt_a

    m = jnp.max(logits, axis=1, keepdims=True)
    e = jnp.exp(logits - m)
    seg_id = jax.lax.broadcasted_iota(jnp.int32, (R, _UB), 0) // cp
    mseg = (seg_id == jax.lax.broadcasted_iota(jnp.int32, (R, _UB), 1)
            ).astype(f32)
    seg_sum = jnp.dot(e, mseg, preferred_element_type=f32)
    denom = jax.lax.dot_general(seg_sum, mseg, (((1,), (1,)), ((), ())),
                                preferred_element_type=f32)
    atten = e / (denom + 1e-30)

    mdiag = (jax.lax.broadcasted_iota(jnp.int32, (_UB, _UB, cp), 1) ==
             jax.lax.broadcasted_iota(jnp.int32, (_UB, _UB, cp), 0)
             ).astype(f32).astype(bf16).reshape(_UB, R)
    a_big = (jnp.broadcast_to(atten.astype(bf16)[:, None, :], (T, _UB, R))
             * mdiag[None]).reshape(T * _UB, R)

    x8 = jnp.dot(a_big, values_bf, preferred_element_type=f32)
    h_ref[0] = x8.reshape(T, _UB, Dh)
    d8 = jnp.dot(x8.astype(bf16), wd_ref[...].astype(bf16),
                 preferred_element_type=f32) + bd_ref[...]
    db_ref[0] = d8.reshape(T, _UB, wd_ref.shape[1])
    p8 = jnp.dot(a_big, S, preferred_element_type=f32)
    ptr_ref[0] = p8.reshape(T, _UB, V1)


def kernel(encoder_out, decoder_in, masks_mat, dec_embed_weight, ooKB_weight,
           Wqa, bqa, Wqs, bqs, Wk, bk, Wd, bd):
    B, T, Eh = encoder_out.shape
    U = decoder_in.shape[1]
    C = masks_mat.shape[2]
    V, Dh = dec_embed_weight.shape
    A = Wk.shape[1]
    J = Wd.shape[1]
    V1 = V + 1
    f32 = jnp.float32

    embs = jnp.concatenate([dec_embed_weight, ooKB_weight], axis=0)

    nu = pl.cdiv(U, _UB)
    u_pad = nu * _UB
    cp = pl.cdiv(C, 8) * 8
    masks_row = masks_mat.astype(jnp.int32)
    masks_pad = jnp.pad(masks_row, ((0, 0), (0, u_pad - U), (0, cp - C)),
                        constant_values=-1)
    masks_pad4 = masks_pad[..., None]
    dec3 = jnp.pad(decoder_in.astype(jnp.int32),
                   ((0, 0), (0, u_pad - U)))[..., None]

    def _half(enc_h, dec_h, maskp_h, maskp4_h):
        Bh = enc_h.shape[0]
        grid = (Bh, nu)
        return pl.pallas_call(
            functools.partial(_main, C=C),
            grid=grid,
        in_specs=[
            pl.BlockSpec((1, T, Eh), lambda b, u: (b, 0, 0)),
            pl.BlockSpec((1, _UB, 1), lambda b, u: (b, u, 0)),
            pl.BlockSpec((1, _UB, cp), lambda b, u: (b, u, 0)),
            pl.BlockSpec((1, _UB, cp, 1), lambda b, u: (b, u, 0, 0)),
            pl.BlockSpec((V1, Dh), lambda b, u: (0, 0)),
            pl.BlockSpec((Eh, A), lambda b, u: (0, 0)),
            pl.BlockSpec((1, A), lambda b, u: (0, 0)),
            pl.BlockSpec((Dh, A), lambda b, u: (0, 0)),
            pl.BlockSpec((1, A), lambda b, u: (0, 0)),
            pl.BlockSpec((Dh, A), lambda b, u: (0, 0)),
            pl.BlockSpec((1, A), lambda b, u: (0, 0)),
            pl.BlockSpec((Dh, J), lambda b, u: (0, 0)),
            pl.BlockSpec((1, J), lambda b, u: (0, 0)),
        ],
        out_specs=[
            pl.BlockSpec((1, T, _UB, V1), lambda b, u: (b, 0, u, 0)),
            pl.BlockSpec((1, T, _UB, Dh), lambda b, u: (b, 0, u, 0)),
            pl.BlockSpec((1, T, _UB, J), lambda b, u: (b, 0, u, 0)),
        ],
        out_shape=(
            jax.ShapeDtypeStruct((Bh, T, U, V1), f32),
            jax.ShapeDtypeStruct((Bh, T, U, Dh), f32),
            jax.ShapeDtypeStruct((Bh, T, U, J), f32),
        ),
        )(enc_h, dec_h, maskp_h, maskp4_h, embs,
          Wqa, bqa.reshape(1, A), Wqs, bqs.reshape(1, A), Wk, bk.reshape(1, A),
          Wd, bd.reshape(1, J))

    bh = B // 2 if B % 2 == 0 and B > 1 else B
    parts = []
    for s in range(0, B, bh):
        parts.append(_half(encoder_out[s:s + bh], dec3[s:s + bh],
                           masks_pad[s:s + bh], masks_pad4[s:s + bh]))
    if len(parts) == 1:
        return parts[0]
    ptr = jnp.concatenate([p[0] for p in parts], axis=0)
    h_ptr = jnp.concatenate([p[1] for p in parts], axis=0)
    dbias = jnp.concatenate([p[2] for p in parts], axis=0)
    return (ptr, h_ptr, dbias)


# speedup vs baseline: 1.2457x; 1.2457x over previous
import functools
import math
import jax
import jax.numpy as jnp
from jax.experimental import pallas as pl

_UB = 8


def _main(enc_ref, dec_ref, idxp_ref, idxc_ref, embs_ref,
          wqa_ref, bqa_ref, wqs_ref, bqs_ref, wk_ref, bk_ref,
          wd_ref, bd_ref, ptr_ref, h_ref, db_ref, *, C):
    T = enc_ref.shape[1]
    cp = idxp_ref.shape[2]
    V1, Dh = embs_ref.shape
    V = V1 - 1
    A = wk_ref.shape[1]
    R = _UB * cp
    bf16 = jnp.bfloat16
    f32 = jnp.float32
    inv_sqrt_a = 1.0 / math.sqrt(A)

    embs = embs_ref[...]
    embs_bf = embs.astype(bf16)

    qac = jnp.dot(enc_ref[0].astype(bf16), wqa_ref[...].astype(bf16),
                  preferred_element_type=f32) + bqa_ref[...]

    dec8 = dec_ref[0]
    onehot_dec = (dec8 == jax.lax.broadcasted_iota(jnp.int32, (_UB, V), 1)
                  ).astype(f32).astype(bf16)
    semantic = jnp.dot(onehot_dec, embs_bf[:V, :],
                       preferred_element_type=f32)
    qse8 = jnp.dot(semantic.astype(bf16), wqs_ref[...].astype(bf16),
                   preferred_element_type=f32) + bqs_ref[...]

    idx_col = idxc_ref[0].reshape(R, 1)
    idx_pad = idxp_ref[0]
    c_col = jax.lax.broadcasted_iota(jnp.int32, (_UB, cp, 1),
                                     1).reshape(R, 1)
    valid = (c_col < C) & (idx_col >= 0)

    iota_v = jax.lax.broadcasted_iota(jnp.int32, (R, V1), 1)
    G_f = jnp.where(idx_col == iota_v, 1.0, 0.0)
    idx_row_exp = jnp.broadcast_to(idx_pad[:, None, :],
                                   (_UB, cp, cp)).reshape(R, cp)
    lane_c = jax.lax.broadcasted_iota(jnp.int32, (R, cp), 1)
    has_later = jnp.any((idx_col == idx_row_exp) & (lane_c > c_col),
                        axis=1, keepdims=True)
    G = G_f.astype(bf16)
    S = jnp.where(has_later | ~valid, 0.0, G_f).astype(bf16)

    values = jnp.dot(G, embs_bf, preferred_element_type=f32)
    values_bf = values.astype(bf16)
    keys = jnp.dot(values_bf, wk_ref[...].astype(bf16),
                   preferred_element_type=f32) + bk_ref[...]

    qse_exp = jnp.broadcast_to(qse8[:, None, :],
                               (_UB, cp, A)).reshape(R, A)
    rowdot = jnp.sum(qse_exp * keys, axis=1, keepdims=True)
    pen = jnp.where(valid, 0.0, -1.0e9)
    k_aug = jnp.concatenate([keys, rowdot, pen],
                            axis=1).astype(bf16)
    ones_t = jnp.full((T, 2), 1.0, dtype=f32)
    q_aug = jnp.concatenate([qac, ones_t], axis=1).astype(bf16)
    qk = jax.lax.dot_general(q_aug, k_aug, (((1,), (1,)), ((), ())),
                             preferred_element_type=f32)
    logits = qk * inv_sqrt_a

    m = jnp.max(logits, axis=1, keepdims=True)
    e = jnp.exp(logits - m)
    seg_id = jax.lax.broadcasted_iota(jnp.int32, (R, _UB), 0) // cp
    mseg = (seg_id == jax.lax.broadcasted_iota(jnp.int32, (R, _UB), 1)
            ).astype(f32)
    seg_sum = jnp.dot(e, mseg, preferred_element_type=f32)
    denom = jax.lax.dot_general(seg_sum, mseg, (((1,), (1,)), ((), ())),
                                preferred_element_type=f32)
    atten = e / (denom + 1e-30)

    mdiag = (jax.lax.broadcasted_iota(jnp.int32, (_UB, _UB, cp), 1) ==
             jax.lax.broadcasted_iota(jnp.int32, (_UB, _UB, cp), 0)
             ).astype(f32).astype(bf16).reshape(_UB, R)
    a_big = (jnp.broadcast_to(atten.astype(bf16)[:, None, :], (T, _UB, R))
             * mdiag[None]).reshape(T * _UB, R)

    x8 = jnp.dot(a_big, values_bf, preferred_element_type=f32)
    h_ref[0] = x8.reshape(T, _UB, Dh)
    d8 = jnp.dot(x8.astype(bf16), wd_ref[...].astype(bf16),
                 preferred_element_type=f32) + bd_ref[...]
    db_ref[0] = d8.reshape(T, _UB, wd_ref.shape[1])
    p8 = jnp.dot(a_big, S, preferred_element_type=f32)
    ptr_ref[0] = p8.reshape(T, _UB, V1)


def kernel(encoder_out, decoder_in, masks_mat, dec_embed_weight, ooKB_weight,
           Wqa, bqa, Wqs, bqs, Wk, bk, Wd, bd):
    B, T, Eh = encoder_out.shape
    U = decoder_in.shape[1]
    C = masks_mat.shape[2]
    V, Dh = dec_embed_weight.shape
    A = Wk.shape[1]
    J = Wd.shape[1]
    V1 = V + 1
    f32 = jnp.float32

    embs = jnp.concatenate([dec_embed_weight, ooKB_weight], axis=0)

    nu = pl.cdiv(U, _UB)
    u_pad = nu * _UB
    cp = pl.cdiv(C, 8) * 8
    masks_row = masks_mat.astype(jnp.int32)
    masks_pad = jnp.pad(masks_row, ((0, 0), (0, u_pad - U), (0, cp - C)),
                        constant_values=-1)
    masks_pad4 = masks_pad[..., None]
    dec3 = jnp.pad(decoder_in.astype(jnp.int32),
                   ((0, 0), (0, u_pad - U)))[..., None]

    def _half(enc_h, dec_h, maskp_h, maskp4_h):
        Bh = enc_h.shape[0]
        grid = (Bh, nu)
        return pl.pallas_call(
            functools.partial(_main, C=C),
            grid=grid,
        in_specs=[
            pl.BlockSpec((1, T, Eh), lambda b, u: (b, 0, 0)),
            pl.BlockSpec((1, _UB, 1), lambda b, u: (b, u, 0)),
            pl.BlockSpec((1, _UB, cp), lambda b, u: (b, u, 0)),
            pl.BlockSpec((1, _UB, cp, 1), lambda b, u: (b, u, 0, 0)),
            pl.BlockSpec((V1, Dh), lambda b, u: (0, 0)),
            pl.BlockSpec((Eh, A), lambda b, u: (0, 0)),
            pl.BlockSpec((1, A), lambda b, u: (0, 0)),
            pl.BlockSpec((Dh, A), lambda b, u: (0, 0)),
            pl.BlockSpec((1, A), lambda b, u: (0, 0)),
            pl.BlockSpec((Dh, A), lambda b, u: (0, 0)),
            pl.BlockSpec((1, A), lambda b, u: (0, 0)),
            pl.BlockSpec((Dh, J), lambda b, u: (0, 0)),
            pl.BlockSpec((1, J), lambda b, u: (0, 0)),
        ],
        out_specs=[
            pl.BlockSpec((1, T, _UB, V1), lambda b, u: (b, 0, u, 0)),
            pl.BlockSpec((1, T, _UB, Dh), lambda b, u: (b, 0, u, 0)),
            pl.BlockSpec((1, T, _UB, J), lambda b, u: (b, 0, u, 0)),
        ],
        out_shape=(
            jax.ShapeDtypeStruct((Bh, T, U, V1), f32),
            jax.ShapeDtypeStruct((Bh, T, U, Dh), f32),
            jax.ShapeDtypeStruct((Bh, T, U, J), f32),
        ),
        )(enc_h, dec_h, maskp_h, maskp4_h, embs,
          Wqa, bqa.reshape(1, A), Wqs, bqs.reshape(1, A), Wk, bk.reshape(1, A),
          Wd, bd.reshape(1, J))

    return _half(encoder_out, dec3, masks_pad, masks_pad4)


# speedup vs baseline: 1.2467x; 1.0008x over previous
import functools
import math
import jax
import jax.numpy as jnp
from jax.experimental import pallas as pl

_UB = 8


def _main(enc_ref, dec_ref, idxp_ref, idxc_ref, embs_ref,
          wqa_ref, bqa_ref, wqs_ref, bqs_ref, wk_ref, bk_ref,
          wd_ref, bd_ref, ptr_ref, h_ref, db_ref, *, C):
    T = enc_ref.shape[1]
    cp = idxp_ref.shape[2]
    V1, Dh = embs_ref.shape
    V = V1 - 1
    A = wk_ref.shape[1]
    R = _UB * cp
    bf16 = jnp.bfloat16
    f32 = jnp.float32
    inv_sqrt_a = 1.0 / math.sqrt(A)

    embs = embs_ref[...]
    embs_bf = embs.astype(bf16)

    qac = jnp.dot(enc_ref[0].astype(bf16), wqa_ref[...].astype(bf16),
                  preferred_element_type=f32) + bqa_ref[...]

    dec8 = dec_ref[0]
    onehot_dec = (dec8 == jax.lax.broadcasted_iota(jnp.int32, (_UB, V), 1)
                  ).astype(f32).astype(bf16)
    semantic = jnp.dot(onehot_dec, embs_bf[:V, :],
                       preferred_element_type=f32)
    qse8 = jnp.dot(semantic.astype(bf16), wqs_ref[...].astype(bf16),
                   preferred_element_type=f32) + bqs_ref[...]

    idx_col = idxc_ref[0].reshape(R, 1)
    idx_pad = idxp_ref[0]
    c_col = jax.lax.broadcasted_iota(jnp.int32, (_UB, cp, 1),
                                     1).reshape(R, 1)
    valid = (c_col < C) & (idx_col >= 0)

    iota_v16 = jax.lax.broadcasted_iota(jnp.int16, (R, V1), 1)
    G = jnp.where(idx_col.astype(jnp.int16) == iota_v16,
                  bf16(1.0), bf16(0.0))
    idx_row_exp = jnp.broadcast_to(idx_pad[:, None, :],
                                   (_UB, cp, cp)).reshape(R, cp)
    lane_c = jax.lax.broadcasted_iota(jnp.int32, (R, cp), 1)
    has_later = jnp.any((idx_col == idx_row_exp) & (lane_c > c_col),
                        axis=1, keepdims=True)
    keep = jnp.where(has_later | ~valid, 0.0, 1.0).astype(bf16)
    S = G * keep

    values = jnp.dot(G, embs_bf, preferred_element_type=f32)
    values_bf = values.astype(bf16)
    keys = jnp.dot(values_bf, wk_ref[...].astype(bf16),
                   preferred_element_type=f32) + bk_ref[...]

    qse_exp = jnp.broadcast_to(qse8[:, None, :],
                               (_UB, cp, A)).reshape(R, A)
    rowdot = jnp.sum(qse_exp * keys, axis=1, keepdims=True)
    pen = jnp.where(valid, 0.0, -1.0e9)
    k_aug = jnp.concatenate([keys, rowdot, pen],
                            axis=1).astype(bf16)
    ones_t = jnp.full((T, 2), 1.0, dtype=f32)
    q_aug = jnp.concatenate([qac, ones_t], axis=1).astype(bf16)
    qk = jax.lax.dot_general(q_aug, k_aug, (((1,), (1,)), ((), ())),
                             preferred_element_type=f32)
    logits = qk * inv_sqrt_a

    m = jnp.max(logits, axis=1, keepdims=True)
    e = jnp.exp(logits - m)
    seg_id = jax.lax.broadcasted_iota(jnp.int32, (R, _UB), 0) // cp
    mseg = (seg_id == jax.lax.broadcasted_iota(jnp.int32, (R, _UB), 1)
            ).astype(f32)
    seg_sum = jnp.dot(e, mseg, preferred_element_type=f32)
    denom = jax.lax.dot_general(seg_sum, mseg, (((1,), (1,)), ((), ())),
                                preferred_element_type=f32)
    atten = e / (denom + 1e-30)

    mdiag = (jax.lax.broadcasted_iota(jnp.int32, (_UB, _UB, cp), 1) ==
             jax.lax.broadcasted_iota(jnp.int32, (_UB, _UB, cp), 0)
             ).astype(f32).astype(bf16).reshape(_UB, R)
    a_big = (jnp.broadcast_to(atten.astype(bf16)[:, None, :], (T, _UB, R))
             * mdiag[None]).reshape(T * _UB, R)

    x8 = jnp.dot(a_big, values_bf, preferred_element_type=f32)
    h_ref[0] = x8.reshape(T, _UB, Dh)
    d8 = jnp.dot(x8.astype(bf16), wd_ref[...].astype(bf16),
                 preferred_element_type=f32) + bd_ref[...]
    db_ref[0] = d8.reshape(T, _UB, wd_ref.shape[1])
    p8 = jnp.dot(a_big, S, preferred_element_type=f32)
    ptr_ref[0] = p8.reshape(T, _UB, V1)


def kernel(encoder_out, decoder_in, masks_mat, dec_embed_weight, ooKB_weight,
           Wqa, bqa, Wqs, bqs, Wk, bk, Wd, bd):
    B, T, Eh = encoder_out.shape
    U = decoder_in.shape[1]
    C = masks_mat.shape[2]
    V, Dh = dec_embed_weight.shape
    A = Wk.shape[1]
    J = Wd.shape[1]
    V1 = V + 1
    f32 = jnp.float32

    embs = jnp.concatenate([dec_embed_weight, ooKB_weight], axis=0)

    nu = pl.cdiv(U, _UB)
    u_pad = nu * _UB
    cp = pl.cdiv(C, 8) * 8
    masks_row = masks_mat.astype(jnp.int32)
    masks_pad = jnp.pad(masks_row, ((0, 0), (0, u_pad - U), (0, cp - C)),
                        constant_values=-1)
    masks_pad4 = masks_pad[..., None]
    dec3 = jnp.pad(decoder_in.astype(jnp.int32),
                   ((0, 0), (0, u_pad - U)))[..., None]

    def _half(enc_h, dec_h, maskp_h, maskp4_h):
        Bh = enc_h.shape[0]
        grid = (Bh, nu)
        return pl.pallas_call(
            functools.partial(_main, C=C),
            grid=grid,
        in_specs=[
            pl.BlockSpec((1, T, Eh), lambda b, u: (b, 0, 0)),
            pl.BlockSpec((1, _UB, 1), lambda b, u: (b, u, 0)),
            pl.BlockSpec((1, _UB, cp), lambda b, u: (b, u, 0)),
            pl.BlockSpec((1, _UB, cp, 1), lambda b, u: (b, u, 0, 0)),
            pl.BlockSpec((V1, Dh), lambda b, u: (0, 0)),
            pl.BlockSpec((Eh, A), lambda b, u: (0, 0)),
            pl.BlockSpec((1, A), lambda b, u: (0, 0)),
            pl.BlockSpec((Dh, A), lambda b, u: (0, 0)),
            pl.BlockSpec((1, A), lambda b, u: (0, 0)),
            pl.BlockSpec((Dh, A), lambda b, u: (0, 0)),
            pl.BlockSpec((1, A), lambda b, u: (0, 0)),
            pl.BlockSpec((Dh, J), lambda b, u: (0, 0)),
            pl.BlockSpec((1, J), lambda b, u: (0, 0)),
        ],
        out_specs=[
            pl.BlockSpec((1, T, _UB, V1), lambda b, u: (b, 0, u, 0)),
            pl.BlockSpec((1, T, _UB, Dh), lambda b, u: (b, 0, u, 0)),
            pl.BlockSpec((1, T, _UB, J), lambda b, u: (b, 0, u, 0)),
        ],
        out_shape=(
            jax.ShapeDtypeStruct((Bh, T, U, V1), f32),
            jax.ShapeDtypeStruct((Bh, T, U, Dh), f32),
            jax.ShapeDtypeStruct((Bh, T, U, J), f32),
        ),
        )(enc_h, dec_h, maskp_h, maskp4_h, embs,
          Wqa, bqa.reshape(1, A), Wqs, bqs.reshape(1, A), Wk, bk.reshape(1, A),
          Wd, bd.reshape(1, J))

    return _half(encoder_out, dec3, masks_pad, masks_pad4)
